# Initial kernel scaffold; baseline (speedup 1.0000x reference)
import functools

import jax
import jax.numpy as jnp
from jax import lax
from jax.experimental import pallas as pl
from jax.experimental.pallas import tpu as pltpu
from jax.experimental.pallas import tpu_sc as plsc

_N = 10000
_E = 160000
_G = 32

_NSC = 2
_NSUB = 16
_CHUNK = 128
_EPAD = 163840
_CPT = _EPAD // (_NSC * _NSUB * _CHUNK)
_NPAD = 10240
_RPT = _NPAD // _NSUB

_mesh = plsc.VectorSubcoreMesh(core_axis_name="c", subcore_axis_name="s")


@functools.partial(
    pl.kernel,
    out_type=jax.ShapeDtypeStruct((_NSC, _NPAD, 16), jnp.float32),
    mesh=_mesh,
    scratch_types=[
        pltpu.VMEM((_CPT, _CHUNK), jnp.int32),
        pltpu.VMEM((_CHUNK, 16), jnp.float32),
        pltpu.VMEM_SHARED((_NPAD, 16), jnp.float32),
        pltpu.SemaphoreType.DMA,
    ],
)
def _sc_degree(dst_hbm, ones_hbm, zeros_hbm, out_hbm, dst_v, ones_v, acc_sh, sem):
    cid = lax.axis_index("c")
    sid = lax.axis_index("s")
    tile = cid * _NSUB + sid
    pltpu.async_copy(zeros_hbm, acc_sh.at[pl.ds(sid * _RPT, _RPT)], sem).wait()
    pltpu.async_copy(ones_hbm, ones_v, sem).wait()
    pltpu.async_copy(dst_hbm.at[pl.ds(tile * _CPT, _CPT)], dst_v, sem).wait()
    plsc.subcore_barrier()

    @pl.loop(0, _CPT)
    def _(j):
        pltpu.sync_copy(ones_v, acc_sh.at[dst_v.at[j]], add=True)

    plsc.subcore_barrier()
    pltpu.sync_copy(
        acc_sh.at[pl.ds(sid * _RPT, _RPT)],
        out_hbm.at[cid, pl.ds(sid * _RPT, _RPT)],
    )


@functools.cache
def _make_sc_segment_sum(dout):
    @functools.partial(
        pl.kernel,
        out_type=jax.ShapeDtypeStruct((_NSC, _NPAD, dout), jnp.float32),
        mesh=_mesh,
        scratch_types=[
            pltpu.VMEM((_CPT, _CHUNK), jnp.int32),
            pltpu.VMEM((_CPT, _CHUNK), jnp.int32),
            pltpu.VMEM((_CHUNK, dout), jnp.float32),
            pltpu.VMEM_SHARED((_NPAD, dout), jnp.float32),
            pltpu.SemaphoreType.DMA,
        ],
    )
    def seg(u_hbm, src_hbm, dst_hbm, zeros_hbm, out_hbm,
            src_v, dst_v, rows_v, acc_sh, sem):
        cid = lax.axis_index("c")
        sid = lax.axis_index("s")
        tile = cid * _NSUB + sid
        pltpu.async_copy(zeros_hbm, acc_sh.at[pl.ds(sid * _RPT, _RPT)], sem).wait()
        pltpu.async_copy(src_hbm.at[pl.ds(tile * _CPT, _CPT)], src_v, sem).wait()
        pltpu.async_copy(dst_hbm.at[pl.ds(tile * _CPT, _CPT)], dst_v, sem).wait()
        plsc.subcore_barrier()

        @pl.loop(0, _CPT)
        def _(j):
            pltpu.async_copy(u_hbm.at[src_v.at[j]], rows_v, sem).wait()
            pltpu.sync_copy(rows_v, acc_sh.at[dst_v.at[j]], add=True)

        plsc.subcore_barrier()
        pltpu.sync_copy(
            acc_sh.at[pl.ds(sid * _RPT, _RPT)],
            out_hbm.at[cid, pl.ds(sid * _RPT, _RPT)],
        )

    return seg


def _tc_first(hp0, hp1, h0, W1):
    def body(hp0_r, hp1_r, h0_r, w_r, dis_o, u_o):
        deg = hp0_r[...] + hp1_r[...] + 1.0
        dis = lax.rsqrt(deg)
        dis_o[...] = dis
        u_o[...] = jnp.dot(h0_r[...], w_r[...],
                           preferred_element_type=jnp.float32) * dis

    return pl.pallas_call(
        body,
        out_shape=(
            jax.ShapeDtypeStruct((_N, 1), jnp.float32),
            jax.ShapeDtypeStruct((_N, W1.shape[1]), jnp.float32),
        ),
    )(hp0, hp1, h0, W1)


def _tc_mid(y0, y1, u, dis, b, W):
    def body(y0_r, y1_r, u_r, dis_r, b_r, w_r, u_o):
        h = jax.nn.relu(dis_r[...] * (y0_r[...] + y1_r[...] + u_r[...]) + b_r[...])
        u_o[...] = jnp.dot(h, w_r[...], preferred_element_type=jnp.float32) * dis_r[...]

    return pl.pallas_call(
        body,
        out_shape=jax.ShapeDtypeStruct((_N, W.shape[1]), jnp.float32),
    )(y0, y1, u, dis, b, W)


def _tc_epilogue(y0, y1, u, dis, b, batch2d, Wl1, bl1, Wl2, bl2):
    def body(y0_r, y1_r, u_r, dis_r, b_r, bat_r, wl1_r, bl1_r, wl2_r, bl2_r, o):
        h = jax.nn.relu(dis_r[...] * (y0_r[...] + y1_r[...] + u_r[...]) + b_r[...])
        gid = lax.broadcasted_iota(jnp.int32, (1, _G), 1)
        onehot = (bat_r[...] == gid).astype(jnp.float32)
        sums = lax.dot_general(onehot, h, (((0,), (0,)), ((), ())),
                               preferred_element_type=jnp.float32)
        cnt = jnp.sum(onehot, axis=0)[:, None]
        g = sums / jnp.clip(cnt, 1.0, None)
        g = jax.nn.relu(jnp.dot(g, wl1_r[...],
                                preferred_element_type=jnp.float32) + bl1_r[...])
        o[...] = jnp.dot(g, wl2_r[...],
                         preferred_element_type=jnp.float32) + bl2_r[...]

    return pl.pallas_call(
        body,
        out_shape=jax.ShapeDtypeStruct((_G, Wl2.shape[1]), jnp.float32),
    )(y0, y1, u, dis, b, batch2d, Wl1, bl1, Wl2, bl2)


def kernel(x, edge_index, batch, W1, b1, W2, b2, W3, b3, W4, b4, W5, b5,
           W6, b6, Wl1, bl1, Wl2, bl2):
    pad = _EPAD - _E
    src = jnp.concatenate([edge_index[0], jnp.zeros((pad,), jnp.int32)])
    dst = jnp.concatenate([edge_index[1], jnp.full((pad,), _N, jnp.int32)])
    src = src.reshape(_EPAD // _CHUNK, _CHUNK)
    dst = dst.reshape(_EPAD // _CHUNK, _CHUNK)

    hist = _sc_degree(dst, jnp.ones((_CHUNK, 16), jnp.float32),
                      jnp.zeros((_RPT, 16), jnp.float32))
    dis, u = _tc_first(hist[0, :_N, :1], hist[1, :_N, :1], x[:, :64], W1)

    layers = [(b1, W2), (b2, W3), (b3, W4), (b4, W5), (b5, W6)]
    for b, Wn in layers:
        dout = u.shape[1]
        y = _make_sc_segment_sum(dout)(
            u, src, dst, jnp.zeros((_RPT, dout), jnp.float32))
        u = _tc_mid(y[0, :_N], y[1, :_N], u, dis, b, Wn)

    y = _make_sc_segment_sum(128)(
        u, src, dst, jnp.zeros((_RPT, 128), jnp.float32))
    return _tc_epilogue(y[0, :_N], y[1, :_N], u, dis, b6,
                        batch.reshape(_N, 1), Wl1, bl1, Wl2, bl2)


# speedup vs baseline: 4.7867x; 4.7867x over previous
import functools

import jax
import jax.numpy as jnp
from jax import lax
from jax.experimental import pallas as pl
from jax.experimental.pallas import tpu as pltpu
from jax.experimental.pallas import tpu_sc as plsc

_N = 10000
_E = 160000
_G = 32
_D = 128

_NSC = 2
_NSUB = 16
_CHUNK = 128
_EPAD = 163840
_CPT = _EPAD // (_NSC * _NSUB * _CHUNK)
_NPAD = 10240
_RPT = _NPAD // _NSUB

_mesh = plsc.VectorSubcoreMesh(core_axis_name="c", subcore_axis_name="s")


@functools.partial(
    pl.kernel,
    out_type=jax.ShapeDtypeStruct((_NSC, _NPAD, _D), jnp.float32),
    mesh=_mesh,
    scratch_types=[
        pltpu.VMEM((_CPT, _CHUNK), jnp.int32),
        pltpu.VMEM((_CHUNK, _D), jnp.float32),
        pltpu.VMEM_SHARED((_NPAD, _D), jnp.float32),
        pltpu.SemaphoreType.DMA,
    ],
)
def _sc_degree(dst_hbm, ones_hbm, zeros_hbm, out_hbm, dst_v, ones_v, acc_sh, sem):
    cid = lax.axis_index("c")
    sid = lax.axis_index("s")
    tile = cid * _NSUB + sid
    pltpu.async_copy(zeros_hbm, acc_sh.at[pl.ds(sid * _RPT, _RPT)], sem).wait()
    pltpu.async_copy(ones_hbm, ones_v, sem).wait()
    pltpu.async_copy(dst_hbm.at[pl.ds(tile * _CPT, _CPT)], dst_v, sem).wait()
    plsc.subcore_barrier()

    @pl.loop(0, _CPT)
    def _(j):
        pltpu.sync_copy(ones_v, acc_sh.at[dst_v.at[j]], add=True)

    plsc.subcore_barrier()
    pltpu.sync_copy(
        acc_sh.at[pl.ds(sid * _RPT, _RPT)],
        out_hbm.at[cid, pl.ds(sid * _RPT, _RPT)],
    )


@functools.partial(
    pl.kernel,
    out_type=jax.ShapeDtypeStruct((_NSC, _NPAD, _D), jnp.float32),
    mesh=_mesh,
    scratch_types=[
        pltpu.VMEM((_CPT, _CHUNK), jnp.int32),
        pltpu.VMEM((_CPT, _CHUNK), jnp.int32),
        pltpu.VMEM((_CHUNK, _D), jnp.float32),
        pltpu.VMEM_SHARED((_NPAD, _D), jnp.float32),
        pltpu.SemaphoreType.DMA,
    ],
)
def _sc_segment_sum(u_hbm, src_hbm, dst_hbm, zeros_hbm, out_hbm,
                    src_v, dst_v, rows_v, acc_sh, sem):
    cid = lax.axis_index("c")
    sid = lax.axis_index("s")
    tile = cid * _NSUB + sid
    pltpu.async_copy(zeros_hbm, acc_sh.at[pl.ds(sid * _RPT, _RPT)], sem).wait()
    pltpu.async_copy(src_hbm.at[pl.ds(tile * _CPT, _CPT)], src_v, sem).wait()
    pltpu.async_copy(dst_hbm.at[pl.ds(tile * _CPT, _CPT)], dst_v, sem).wait()
    plsc.subcore_barrier()

    @pl.loop(0, _CPT)
    def _(j):
        pltpu.async_copy(u_hbm.at[src_v.at[j]], rows_v, sem).wait()
        pltpu.sync_copy(rows_v, acc_sh.at[dst_v.at[j]], add=True)

    plsc.subcore_barrier()
    pltpu.sync_copy(
        acc_sh.at[pl.ds(sid * _RPT, _RPT)],
        out_hbm.at[cid, pl.ds(sid * _RPT, _RPT)],
    )


def _tc_first(hp0, hp1, x, W1):
    def body(hp0_r, hp1_r, x_r, w_r, dis_o, u_o):
        deg = hp0_r[...] + hp1_r[...] + 1.0
        dis = lax.rsqrt(deg)
        dis_o[...] = dis
        u_o[...] = jnp.dot(x_r[...], w_r[...],
                           preferred_element_type=jnp.float32) * dis

    return pl.pallas_call(
        body,
        out_shape=(
            jax.ShapeDtypeStruct((_N, 1), jnp.float32),
            jax.ShapeDtypeStruct((_N, _D), jnp.float32),
        ),
    )(hp0, hp1, x, W1)


def _tc_mid(y0, y1, u, dis, b, W):
    def body(y0_r, y1_r, u_r, dis_r, b_r, w_r, u_o):
        h = jax.nn.relu(dis_r[...] * (y0_r[...] + y1_r[...] + u_r[...]) + b_r[...])
        u_o[...] = jnp.dot(h, w_r[...], preferred_element_type=jnp.float32) * dis_r[...]

    return pl.pallas_call(
        body,
        out_shape=jax.ShapeDtypeStruct((_N, _D), jnp.float32),
    )(y0, y1, u, dis, b, W)


def _tc_epilogue(y0, y1, u, dis, b, batch2d, Wl1, bl1, Wl2, bl2):
    def body(y0_r, y1_r, u_r, dis_r, b_r, bat_r, wl1_r, bl1_r, wl2_r, bl2_r, o):
        h = jax.nn.relu(dis_r[...] * (y0_r[...] + y1_r[...] + u_r[...]) + b_r[...])
        gid = lax.broadcasted_iota(jnp.int32, (1, _G), 1)
        onehot = (bat_r[...] == gid).astype(jnp.float32)
        sums = lax.dot_general(onehot, h, (((0,), (0,)), ((), ())),
                               preferred_element_type=jnp.float32)
        cnt = jnp.sum(onehot, axis=0)[:, None]
        g = sums / jnp.clip(cnt, 1.0, None)
        g = jax.nn.relu(jnp.dot(g, wl1_r[...],
                                preferred_element_type=jnp.float32) + bl1_r[...])
        o[...] = jnp.dot(g, wl2_r[...],
                         preferred_element_type=jnp.float32) + bl2_r[...]

    return pl.pallas_call(
        body,
        out_shape=jax.ShapeDtypeStruct((_G, Wl2.shape[1]), jnp.float32),
    )(y0, y1, u, dis, b, batch2d, Wl1, bl1, Wl2, bl2)


def _pad_w(W):
    return jnp.zeros((_D, _D), jnp.float32).at[:W.shape[0], :W.shape[1]].set(W)


def _pad_b(b):
    return jnp.zeros((_D,), jnp.float32).at[:b.shape[0]].set(b)


def kernel(x, edge_index, batch, W1, b1, W2, b2, W3, b3, W4, b4, W5, b5,
           W6, b6, Wl1, bl1, Wl2, bl2):
    pad = _EPAD - _E
    src = jnp.concatenate([edge_index[0], jnp.zeros((pad,), jnp.int32)])
    dst = jnp.concatenate([edge_index[1], jnp.full((pad,), _N, jnp.int32)])
    src = src.reshape(_EPAD // _CHUNK, _CHUNK)
    dst = dst.reshape(_EPAD // _CHUNK, _CHUNK)
    zeros = jnp.zeros((_RPT, _D), jnp.float32)

    hist = _sc_degree(dst, jnp.ones((_CHUNK, _D), jnp.float32), zeros)
    W1p = jnp.zeros((_D, _D), jnp.float32).at[:64, :W1.shape[1]].set(W1)
    dis, u = _tc_first(hist[0, :_N, :1], hist[1, :_N, :1], x, W1p)

    layers = [(b1, W2), (b2, W3), (b3, W4), (b4, W5), (b5, W6)]
    for b, Wn in layers:
        y = _sc_segment_sum(u, src, dst, zeros)
        u = _tc_mid(y[0, :_N], y[1, :_N], u, dis, _pad_b(b), _pad_w(Wn))

    y = _sc_segment_sum(u, src, dst, zeros)
    return _tc_epilogue(y[0, :_N], y[1, :_N], u, dis, _pad_b(b6),
                        batch.reshape(_N, 1), Wl1, bl1, Wl2, bl2)


# speedup vs baseline: 5.2937x; 1.1059x over previous
import functools

import jax
import jax.numpy as jnp
from jax import lax
from jax.experimental import pallas as pl
from jax.experimental.pallas import tpu as pltpu
from jax.experimental.pallas import tpu_sc as plsc

_N = 10000
_E = 160000
_G = 32
_D = 128

_NSC = 2
_NSUB = 16
_CHUNK = 128
_EPAD = 163840
_CPT = _EPAD // (_NSC * _NSUB * _CHUNK)
_NPAD = 10240
_RPT = _NPAD // _NSUB

_mesh = plsc.VectorSubcoreMesh(core_axis_name="c", subcore_axis_name="s")


@functools.partial(
    pl.kernel,
    out_type=jax.ShapeDtypeStruct((_NSC, _NPAD, _D), jnp.float32),
    mesh=_mesh,
    scratch_types=[
        pltpu.VMEM((_CPT, _CHUNK), jnp.int32),
        pltpu.VMEM((_CHUNK, _D), jnp.float32),
        pltpu.VMEM_SHARED((_NPAD, _D), jnp.float32),
        pltpu.SemaphoreType.DMA,
    ],
)
def _sc_degree(dst_hbm, ones_hbm, zeros_hbm, out_hbm, dst_v, ones_v, acc_sh, sem):
    cid = lax.axis_index("c")
    sid = lax.axis_index("s")
    tile = cid * _NSUB + sid
    pltpu.async_copy(zeros_hbm, acc_sh.at[pl.ds(sid * _RPT, _RPT)], sem).wait()
    pltpu.async_copy(ones_hbm, ones_v, sem).wait()
    pltpu.async_copy(dst_hbm.at[pl.ds(tile * _CPT, _CPT)], dst_v, sem).wait()
    plsc.subcore_barrier()

    @pl.loop(0, _CPT)
    def _(j):
        pltpu.sync_copy(ones_v, acc_sh.at[dst_v.at[j]], add=True)

    plsc.subcore_barrier()
    pltpu.sync_copy(
        acc_sh.at[pl.ds(sid * _RPT, _RPT)],
        out_hbm.at[cid, pl.ds(sid * _RPT, _RPT)],
    )


@functools.partial(
    pl.kernel,
    out_type=jax.ShapeDtypeStruct((_NSC, _NPAD, _D), jnp.float32),
    mesh=_mesh,
    scratch_types=[
        pltpu.VMEM((_CPT, _CHUNK), jnp.int32),
        pltpu.VMEM((_CPT, _CHUNK), jnp.int32),
        pltpu.VMEM((_CHUNK, _D), jnp.float32),
        pltpu.VMEM((_CHUNK, _D), jnp.float32),
        pltpu.SemaphoreType.DMA,
        pltpu.SemaphoreType.DMA,
        pltpu.SemaphoreType.DMA,
        pltpu.VMEM_SHARED((_NPAD, _D), jnp.float32),
    ],
)
def _sc_segment_sum(u_hbm, src_hbm, dst_hbm, zeros_hbm, out_hbm,
                    src_v, dst_v, buf_a, buf_b, sem, sem_a, sem_b, acc_sh):
    cid = lax.axis_index("c")
    sid = lax.axis_index("s")
    tile = cid * _NSUB + sid
    pltpu.async_copy(src_hbm.at[pl.ds(tile * _CPT, _CPT)], src_v, sem).wait()
    pltpu.async_copy(dst_hbm.at[pl.ds(tile * _CPT, _CPT)], dst_v, sem).wait()
    pltpu.async_copy(u_hbm.at[src_v.at[0]], buf_a, sem_a)
    pltpu.sync_copy(zeros_hbm, acc_sh.at[pl.ds(sid * _RPT, _RPT)])
    plsc.subcore_barrier()

    @pl.loop(0, _CPT, step=2)
    def _(j):
        pltpu.async_copy(u_hbm.at[src_v.at[j + 1]], buf_b, sem_b)
        pltpu.make_async_copy(u_hbm.at[src_v.at[j]], buf_a, sem_a).wait()
        pltpu.sync_copy(buf_a, acc_sh.at[dst_v.at[j]], add=True)

        @pl.when(j + 2 < _CPT)
        def _():
            pltpu.async_copy(u_hbm.at[src_v.at[j + 2]], buf_a, sem_a)

        pltpu.make_async_copy(u_hbm.at[src_v.at[j + 1]], buf_b, sem_b).wait()
        pltpu.sync_copy(buf_b, acc_sh.at[dst_v.at[j + 1]], add=True)

    plsc.subcore_barrier()
    pltpu.sync_copy(
        acc_sh.at[pl.ds(sid * _RPT, _RPT)],
        out_hbm.at[cid, pl.ds(sid * _RPT, _RPT)],
    )


def _tc_first(hp0, hp1, x, W1):
    def body(hp0_r, hp1_r, x_r, w_r, dis_o, u_o):
        deg = hp0_r[...] + hp1_r[...] + 1.0
        dis = lax.rsqrt(deg)
        dis_o[...] = dis
        u_o[...] = jnp.dot(x_r[...], w_r[...],
                           preferred_element_type=jnp.float32) * dis

    return pl.pallas_call(
        body,
        out_shape=(
            jax.ShapeDtypeStruct((_N, 1), jnp.float32),
            jax.ShapeDtypeStruct((_N, _D), jnp.float32),
        ),
    )(hp0, hp1, x, W1)


def _tc_mid(y0, y1, u, dis, b, W):
    def body(y0_r, y1_r, u_r, dis_r, b_r, w_r, u_o):
        h = jax.nn.relu(dis_r[...] * (y0_r[...] + y1_r[...] + u_r[...]) + b_r[...])
        u_o[...] = jnp.dot(h, w_r[...], preferred_element_type=jnp.float32) * dis_r[...]

    return pl.pallas_call(
        body,
        out_shape=jax.ShapeDtypeStruct((_N, _D), jnp.float32),
    )(y0, y1, u, dis, b, W)


def _tc_epilogue(y0, y1, u, dis, b, batch2d, Wl1, bl1, Wl2, bl2):
    def body(y0_r, y1_r, u_r, dis_r, b_r, bat_r, wl1_r, bl1_r, wl2_r, bl2_r, o):
        h = jax.nn.relu(dis_r[...] * (y0_r[...] + y1_r[...] + u_r[...]) + b_r[...])
        gid = lax.broadcasted_iota(jnp.int32, (1, _G), 1)
        onehot = (bat_r[...] == gid).astype(jnp.float32)
        sums = lax.dot_general(onehot, h, (((0,), (0,)), ((), ())),
                               preferred_element_type=jnp.float32)
        cnt = jnp.sum(onehot, axis=0)[:, None]
        g = sums / jnp.clip(cnt, 1.0, None)
        g = jax.nn.relu(jnp.dot(g, wl1_r[...],
                                preferred_element_type=jnp.float32) + bl1_r[...])
        o[...] = jnp.dot(g, wl2_r[...],
                         preferred_element_type=jnp.float32) + bl2_r[...]

    return pl.pallas_call(
        body,
        out_shape=jax.ShapeDtypeStruct((_G, Wl2.shape[1]), jnp.float32),
    )(y0, y1, u, dis, b, batch2d, Wl1, bl1, Wl2, bl2)


def _pad_w(W):
    return jnp.zeros((_D, _D), jnp.float32).at[:W.shape[0], :W.shape[1]].set(W)


def _pad_b(b):
    return jnp.zeros((_D,), jnp.float32).at[:b.shape[0]].set(b)


def kernel(x, edge_index, batch, W1, b1, W2, b2, W3, b3, W4, b4, W5, b5,
           W6, b6, Wl1, bl1, Wl2, bl2):
    pad = _EPAD - _E
    src = jnp.concatenate([edge_index[0], jnp.zeros((pad,), jnp.int32)])
    dst = jnp.concatenate([edge_index[1], jnp.full((pad,), _N, jnp.int32)])
    src = src.reshape(_EPAD // _CHUNK, _CHUNK)
    dst = dst.reshape(_EPAD // _CHUNK, _CHUNK)
    zeros = jnp.zeros((_RPT, _D), jnp.float32)

    hist = _sc_degree(dst, jnp.ones((_CHUNK, _D), jnp.float32), zeros)
    W1p = jnp.zeros((_D, _D), jnp.float32).at[:64, :W1.shape[1]].set(W1)
    dis, u = _tc_first(hist[0, :_N, :1], hist[1, :_N, :1], x, W1p)

    layers = [(b1, W2), (b2, W3), (b3, W4), (b4, W5), (b5, W6)]
    for b, Wn in layers:
        y = _sc_segment_sum(u, src, dst, zeros)
        u = _tc_mid(y[0, :_N], y[1, :_N], u, dis, _pad_b(b), _pad_w(Wn))

    y = _sc_segment_sum(u, src, dst, zeros)
    return _tc_epilogue(y[0, :_N], y[1, :_N], u, dis, _pad_b(b6),
                        batch.reshape(_N, 1), Wl1, bl1, Wl2, bl2)


# speedup vs baseline: 5.3406x; 1.0089x over previous
import functools

import jax
import jax.numpy as jnp
from jax import lax
from jax.experimental import pallas as pl
from jax.experimental.pallas import tpu as pltpu
from jax.experimental.pallas import tpu_sc as plsc

_N = 10000
_E = 160000
_G = 32
_D = 128

_NSC = 2
_NSUB = 16
_CHUNK = 128
_EPAD = 163840
_CPT = _EPAD // (_NSC * _NSUB * _CHUNK)
_NPAD = 10240
_RPT = _NPAD // _NSUB

_mesh = plsc.VectorSubcoreMesh(core_axis_name="c", subcore_axis_name="s")


@functools.partial(
    pl.kernel,
    out_type=jax.ShapeDtypeStruct((_NSC, _NPAD, _D), jnp.float32),
    mesh=_mesh,
    scratch_types=[
        pltpu.VMEM((_CPT, _CHUNK), jnp.int32),
        pltpu.VMEM((_CHUNK, _D), jnp.float32),
        pltpu.SemaphoreType.DMA,
        pltpu.SemaphoreType.DMA,
        pltpu.VMEM_SHARED((_NPAD, _D), jnp.float32),
    ],
)
def _sc_degree(dst_hbm, out_hbm, dst_v, buf, sem, ssem, acc_sh):
    cid = lax.axis_index("c")
    sid = lax.axis_index("s")
    tile = cid * _NSUB + sid
    pltpu.async_copy(dst_hbm.at[pl.ds(tile * _CPT, _CPT)], dst_v, sem)

    @pl.loop(0, _CHUNK, unroll=4)
    def _(r):
        for k in range(_D // 16):
            buf[r, pl.ds(k * 16, 16)] = jnp.zeros((16,), jnp.float32)

    @pl.loop(0, _RPT, step=_CHUNK)
    def _(r):
        pltpu.sync_copy(buf, acc_sh.at[pl.ds(sid * _RPT + r, _CHUNK)])

    @pl.loop(0, _CHUNK, unroll=8)
    def _(r):
        buf[r, pl.ds(0, 16)] = jnp.full((16,), 1.0, jnp.float32)

    pltpu.make_async_copy(dst_hbm.at[pl.ds(tile * _CPT, _CPT)], dst_v, sem).wait()
    plsc.subcore_barrier()

    pltpu.async_copy(buf, acc_sh.at[dst_v.at[0]], ssem, add=True)

    @pl.loop(1, _CPT)
    def _(j):
        pltpu.async_copy(buf, acc_sh.at[dst_v.at[j]], ssem, add=True)
        pltpu.make_async_copy(buf, acc_sh.at[dst_v.at[j - 1]], ssem).wait()

    pltpu.make_async_copy(buf, acc_sh.at[dst_v.at[_CPT - 1]], ssem).wait()
    plsc.subcore_barrier()
    pltpu.sync_copy(
        acc_sh.at[pl.ds(sid * _RPT, _RPT)],
        out_hbm.at[cid, pl.ds(sid * _RPT, _RPT)],
    )


@functools.partial(
    pl.kernel,
    out_type=jax.ShapeDtypeStruct((_NSC, _NPAD, _D), jnp.float32),
    mesh=_mesh,
    scratch_types=[
        pltpu.VMEM((_CPT, _CHUNK), jnp.int32),
        pltpu.VMEM((_CPT, _CHUNK), jnp.int32),
        [pltpu.VMEM((_CHUNK, _D), jnp.float32)] * 2,
        [pltpu.SemaphoreType.DMA] * 2,
        [pltpu.SemaphoreType.DMA] * 2,
        pltpu.SemaphoreType.DMA,
        pltpu.VMEM_SHARED((_NPAD, _D), jnp.float32),
    ],
)
def _sc_segment_sum(u_hbm, src_hbm, dst_hbm, out_hbm,
                    src_v, dst_v, bufs, gsems, ssems, sem, acc_sh):
    cid = lax.axis_index("c")
    sid = lax.axis_index("s")
    tile = cid * _NSUB + sid
    pltpu.async_copy(src_hbm.at[pl.ds(tile * _CPT, _CPT)], src_v, sem).wait()
    pltpu.async_copy(u_hbm.at[src_v.at[0]], bufs[0], gsems[0])
    pltpu.async_copy(dst_hbm.at[pl.ds(tile * _CPT, _CPT)], dst_v, sem)

    @pl.loop(0, _CHUNK, unroll=4)
    def _(r):
        for k in range(_D // 16):
            bufs[1][r, pl.ds(k * 16, 16)] = jnp.zeros((16,), jnp.float32)

    @pl.loop(0, _RPT, step=_CHUNK)
    def _(r):
        pltpu.sync_copy(bufs[1], acc_sh.at[pl.ds(sid * _RPT + r, _CHUNK)])

    pltpu.make_async_copy(dst_hbm.at[pl.ds(tile * _CPT, _CPT)], dst_v, sem).wait()
    pltpu.async_copy(u_hbm.at[src_v.at[1]], bufs[1], gsems[1])
    plsc.subcore_barrier()

    @pl.loop(0, _CPT, step=2)
    def _(j):
        for t in range(2):
            pltpu.make_async_copy(
                u_hbm.at[src_v.at[j + t]], bufs[t], gsems[t]).wait()
            pltpu.async_copy(bufs[t], acc_sh.at[dst_v.at[j + t]], ssems[t],
                             add=True)
        for t in range(2):
            @pl.when(j + t + 2 < _CPT)
            def _(t=t, j=j):
                pltpu.make_async_copy(
                    bufs[t], acc_sh.at[dst_v.at[j + t]], ssems[t]).wait()
                pltpu.async_copy(
                    u_hbm.at[src_v.at[j + t + 2]], bufs[t], gsems[t])

    for t in range(2):
        pltpu.make_async_copy(bufs[t], acc_sh.at[dst_v.at[0]], ssems[t]).wait()
    plsc.subcore_barrier()
    pltpu.sync_copy(
        acc_sh.at[pl.ds(sid * _RPT, _RPT)],
        out_hbm.at[cid, pl.ds(sid * _RPT, _RPT)],
    )


def _tc_first(hp0, hp1, x, W1):
    def body(hp0_r, hp1_r, x_r, w_r, dis_o, u_o):
        deg = hp0_r[...] + hp1_r[...] + 1.0
        dis = lax.rsqrt(deg)
        dis_o[...] = dis
        u_o[...] = jnp.dot(x_r[...], w_r[...],
                           preferred_element_type=jnp.float32) * dis

    return pl.pallas_call(
        body,
        out_shape=(
            jax.ShapeDtypeStruct((_N, 1), jnp.float32),
            jax.ShapeDtypeStruct((_N, _D), jnp.float32),
        ),
    )(hp0, hp1, x, W1)


def _tc_mid(y0, y1, u, dis, b, W):
    def body(y0_r, y1_r, u_r, dis_r, b_r, w_r, u_o):
        h = jax.nn.relu(dis_r[...] * (y0_r[...] + y1_r[...] + u_r[...]) + b_r[...])
        u_o[...] = jnp.dot(h, w_r[...], preferred_element_type=jnp.float32) * dis_r[...]

    return pl.pallas_call(
        body,
        out_shape=jax.ShapeDtypeStruct((_N, _D), jnp.float32),
    )(y0, y1, u, dis, b, W)


def _tc_epilogue(y0, y1, u, dis, b, batch2d, Wl1, bl1, Wl2, bl2):
    def body(y0_r, y1_r, u_r, dis_r, b_r, bat_r, wl1_r, bl1_r, wl2_r, bl2_r, o):
        h = jax.nn.relu(dis_r[...] * (y0_r[...] + y1_r[...] + u_r[...]) + b_r[...])
        gid = lax.broadcasted_iota(jnp.int32, (1, _G), 1)
        onehot = (bat_r[...] == gid).astype(jnp.float32)
        sums = lax.dot_general(onehot, h, (((0,), (0,)), ((), ())),
                               preferred_element_type=jnp.float32)
        cnt = jnp.sum(onehot, axis=0)[:, None]
        g = sums / jnp.clip(cnt, 1.0, None)
        g = jax.nn.relu(jnp.dot(g, wl1_r[...],
                                preferred_element_type=jnp.float32) + bl1_r[...])
        o[...] = jnp.dot(g, wl2_r[...],
                         preferred_element_type=jnp.float32) + bl2_r[...]

    return pl.pallas_call(
        body,
        out_shape=jax.ShapeDtypeStruct((_G, Wl2.shape[1]), jnp.float32),
    )(y0, y1, u, dis, b, batch2d, Wl1, bl1, Wl2, bl2)


def _pad_w(W):
    return jnp.zeros((_D, _D), jnp.float32).at[:W.shape[0], :W.shape[1]].set(W)


def _pad_b(b):
    return jnp.zeros((_D,), jnp.float32).at[:b.shape[0]].set(b)


def kernel(x, edge_index, batch, W1, b1, W2, b2, W3, b3, W4, b4, W5, b5,
           W6, b6, Wl1, bl1, Wl2, bl2):
    pad = _EPAD - _E
    src = jnp.concatenate([edge_index[0], jnp.zeros((pad,), jnp.int32)])
    dst = jnp.concatenate([edge_index[1], jnp.full((pad,), _N, jnp.int32)])
    src = src.reshape(_EPAD // _CHUNK, _CHUNK)
    dst = dst.reshape(_EPAD // _CHUNK, _CHUNK)
    hist = _sc_degree(dst)
    W1p = jnp.zeros((_D, _D), jnp.float32).at[:64, :W1.shape[1]].set(W1)
    dis, u = _tc_first(hist[0, :_N, :1], hist[1, :_N, :1], x, W1p)

    layers = [(b1, W2), (b2, W3), (b3, W4), (b4, W5), (b5, W6)]
    for b, Wn in layers:
        y = _sc_segment_sum(u, src, dst)
        u = _tc_mid(y[0, :_N], y[1, :_N], u, dis, _pad_b(b), _pad_w(Wn))

    y = _sc_segment_sum(u, src, dst)
    return _tc_epilogue(y[0, :_N], y[1, :_N], u, dis, _pad_b(b6),
                        batch.reshape(_N, 1), Wl1, bl1, Wl2, bl2)


# speedup vs baseline: 14.5530x; 2.7250x over previous
import functools

import jax
import jax.numpy as jnp
from jax import lax
from jax.experimental import pallas as pl
from jax.experimental.pallas import tpu as pltpu
from jax.experimental.pallas import tpu_sc as plsc

_N = 10000
_E = 160000
_G = 32
_D = 128

_NSC = 2
_NSUB = 16
_CHUNK = 128
_EPAD = 163840
_CPT = _EPAD // (_NSC * _NSUB * _CHUNK)
_NPAD = 10240
_RPT = _NPAD // _NSUB

_mesh = plsc.VectorSubcoreMesh(core_axis_name="c", subcore_axis_name="s")


@functools.partial(
    pl.kernel,
    out_type=jax.ShapeDtypeStruct((_NSC, _NPAD, _D), jnp.float32),
    mesh=_mesh,
    scratch_types=[
        pltpu.VMEM((_CPT, _CHUNK), jnp.int32),
        pltpu.VMEM((_CHUNK, _D), jnp.float32),
        pltpu.SemaphoreType.DMA,
        pltpu.SemaphoreType.DMA,
        pltpu.VMEM_SHARED((_NPAD, _D), jnp.float32),
    ],
)
def _sc_degree(dst_hbm, out_hbm, dst_v, buf, sem, ssem, acc_sh):
    cid = lax.axis_index("c")
    sid = lax.axis_index("s")
    tile = cid * _NSUB + sid
    pltpu.async_copy(dst_hbm.at[pl.ds(tile * _CPT, _CPT)], dst_v, sem)

    @pl.loop(0, _CHUNK, unroll=4)
    def _(r):
        for k in range(_D // 16):
            buf[r, pl.ds(k * 16, 16)] = jnp.zeros((16,), jnp.float32)

    @pl.loop(0, _RPT, step=_CHUNK)
    def _(r):
        pltpu.sync_copy(buf, acc_sh.at[pl.ds(sid * _RPT + r, _CHUNK)])

    @pl.loop(0, _CHUNK, unroll=8)
    def _(r):
        buf[r, pl.ds(0, 16)] = jnp.full((16,), 1.0, jnp.float32)

    pltpu.make_async_copy(dst_hbm.at[pl.ds(tile * _CPT, _CPT)], dst_v, sem).wait()
    plsc.subcore_barrier()

    pltpu.async_copy(buf, acc_sh.at[dst_v.at[0]], ssem, add=True)

    @pl.loop(1, _CPT)
    def _(j):
        pltpu.async_copy(buf, acc_sh.at[dst_v.at[j]], ssem, add=True)
        pltpu.make_async_copy(buf, acc_sh.at[dst_v.at[j - 1]], ssem).wait()

    pltpu.make_async_copy(buf, acc_sh.at[dst_v.at[_CPT - 1]], ssem).wait()
    plsc.subcore_barrier()
    pltpu.sync_copy(
        acc_sh.at[pl.ds(sid * _RPT, _RPT)],
        out_hbm.at[cid, pl.ds(sid * _RPT, _RPT)],
    )


@functools.partial(
    pl.kernel,
    out_type=jax.ShapeDtypeStruct((_NSC, _NPAD, _D), jnp.float32),
    mesh=_mesh,
    scratch_types=[
        pltpu.VMEM((_CPT, _CHUNK), jnp.int32),
        pltpu.VMEM((_CPT, _CHUNK), jnp.int32),
        [pltpu.VMEM((_CHUNK, _D), jnp.float32)] * 2,
        [pltpu.SemaphoreType.DMA] * 2,
        [pltpu.SemaphoreType.DMA] * 2,
        pltpu.SemaphoreType.DMA,
        pltpu.VMEM_SHARED((_NPAD, _D), jnp.float32),
    ],
)
def _sc_segment_sum(u_hbm, src_hbm, dst_hbm, out_hbm,
                    src_v, dst_v, bufs, gsems, ssems, sem, acc_sh):
    cid = lax.axis_index("c")
    sid = lax.axis_index("s")
    tile = cid * _NSUB + sid
    pltpu.async_copy(src_hbm.at[pl.ds(tile * _CPT, _CPT)], src_v, sem).wait()
    pltpu.async_copy(u_hbm.at[src_v.at[0]], bufs[0], gsems[0])
    pltpu.async_copy(dst_hbm.at[pl.ds(tile * _CPT, _CPT)], dst_v, sem)

    @pl.loop(0, _CHUNK, unroll=4)
    def _(r):
        for k in range(_D // 16):
            bufs[1][r, pl.ds(k * 16, 16)] = jnp.zeros((16,), jnp.float32)

    @pl.loop(0, _RPT, step=_CHUNK)
    def _(r):
        pltpu.sync_copy(bufs[1], acc_sh.at[pl.ds(sid * _RPT + r, _CHUNK)])

    pltpu.make_async_copy(dst_hbm.at[pl.ds(tile * _CPT, _CPT)], dst_v, sem).wait()
    pltpu.async_copy(u_hbm.at[src_v.at[1]], bufs[1], gsems[1])
    plsc.subcore_barrier()

    @pl.loop(0, _CPT, step=2)
    def _(j):
        for t in range(2):
            pltpu.make_async_copy(
                u_hbm.at[src_v.at[j + t]], bufs[t], gsems[t]).wait()
            pltpu.async_copy(bufs[t], acc_sh.at[dst_v.at[j + t]], ssems[t],
                             add=True)
        for t in range(2):
            @pl.when(j + t + 2 < _CPT)
            def _(t=t, j=j):
                pltpu.make_async_copy(
                    bufs[t], acc_sh.at[dst_v.at[j + t]], ssems[t]).wait()
                pltpu.async_copy(
                    u_hbm.at[src_v.at[j + t + 2]], bufs[t], gsems[t])

    for t in range(2):
        pltpu.make_async_copy(bufs[t], acc_sh.at[dst_v.at[0]], ssems[t]).wait()
    plsc.subcore_barrier()
    pltpu.sync_copy(
        acc_sh.at[pl.ds(sid * _RPT, _RPT)],
        out_hbm.at[cid, pl.ds(sid * _RPT, _RPT)],
    )


def _tc_first(hp0, hp1, x, W1):
    def body(hp0_r, hp1_r, x_r, w_r, dis_o, u_o):
        deg = hp0_r[...] + hp1_r[...] + 1.0
        dis = lax.rsqrt(deg)
        dis_o[...] = dis
        u_o[...] = jnp.dot(x_r[...], w_r[...],
                           preferred_element_type=jnp.float32) * dis

    return pl.pallas_call(
        body,
        out_shape=(
            jax.ShapeDtypeStruct((_N, 1), jnp.float32),
            jax.ShapeDtypeStruct((_N, _D), jnp.float32),
        ),
    )(hp0, hp1, x, W1)


def _tc_mid(y0, y1, u, dis, b, W):
    def body(y0_r, y1_r, u_r, dis_r, b_r, w_r, u_o):
        h = jax.nn.relu(dis_r[...] * (y0_r[...] + y1_r[...] + u_r[...]) + b_r[...])
        u_o[...] = jnp.dot(h, w_r[...], preferred_element_type=jnp.float32) * dis_r[...]

    return pl.pallas_call(
        body,
        out_shape=jax.ShapeDtypeStruct((_N, _D), jnp.float32),
    )(y0, y1, u, dis, b, W)


def _tc_epilogue(y0, y1, u, dis, b, batch2d, Wl1, bl1, Wl2, bl2):
    def body(y0_r, y1_r, u_r, dis_r, b_r, bat_r, wl1_r, bl1_r, wl2_r, bl2_r, o):
        h = jax.nn.relu(dis_r[...] * (y0_r[...] + y1_r[...] + u_r[...]) + b_r[...])
        gid = lax.broadcasted_iota(jnp.int32, (1, _G), 1)
        onehot = (bat_r[...] == gid).astype(jnp.float32)
        sums = lax.dot_general(onehot, h, (((0,), (0,)), ((), ())),
                               preferred_element_type=jnp.float32)
        cnt = jnp.sum(onehot, axis=0)[:, None]
        g = sums / jnp.clip(cnt, 1.0, None)
        g = jax.nn.relu(jnp.dot(g, wl1_r[...],
                                preferred_element_type=jnp.float32) + bl1_r[...])
        o[...] = jnp.dot(g, wl2_r[...],
                         preferred_element_type=jnp.float32) + bl2_r[...]

    return pl.pallas_call(
        body,
        out_shape=jax.ShapeDtypeStruct((_G, Wl2.shape[1]), jnp.float32),
    )(y0, y1, u, dis, b, batch2d, Wl1, bl1, Wl2, bl2)


def _pad_w(W):
    return jnp.zeros((_D, _D), jnp.float32).at[:W.shape[0], :W.shape[1]].set(W)


def _pad_b(b):
    return jnp.zeros((_D,), jnp.float32).at[:b.shape[0]].set(b)


def kernel(x, edge_index, batch, W1, b1, W2, b2, W3, b3, W4, b4, W5, b5,
           W6, b6, Wl1, bl1, Wl2, bl2):
    pad = _EPAD - _E
    pad_iota = jnp.arange(pad, dtype=jnp.int32)
    src = jnp.concatenate([edge_index[0], pad_iota % _N])
    dst = jnp.concatenate([edge_index[1], _N + pad_iota % (_NPAD - _N)])
    src = src.reshape(_EPAD // _CHUNK, _CHUNK)
    dst = dst.reshape(_EPAD // _CHUNK, _CHUNK)
    hist = _sc_degree(dst)
    W1p = jnp.zeros((_D, _D), jnp.float32).at[:64, :W1.shape[1]].set(W1)
    dis, u = _tc_first(hist[0, :_N, :1], hist[1, :_N, :1], x, W1p)

    layers = [(b1, W2), (b2, W3), (b3, W4), (b4, W5), (b5, W6)]
    for b, Wn in layers:
        y = _sc_segment_sum(u, src, dst)
        u = _tc_mid(y[0, :_N], y[1, :_N], u, dis, _pad_b(b), _pad_w(Wn))

    y = _sc_segment_sum(u, src, dst)
    return _tc_epilogue(y[0, :_N], y[1, :_N], u, dis, _pad_b(b6),
                        batch.reshape(_N, 1), Wl1, bl1, Wl2, bl2)


# speedup vs baseline: 15.7452x; 1.0819x over previous
import functools

import jax
import jax.numpy as jnp
from jax import lax
from jax.experimental import pallas as pl
from jax.experimental.pallas import tpu as pltpu
from jax.experimental.pallas import tpu_sc as plsc

_N = 10000
_E = 160000
_G = 32
_D = 128

_NSC = 2
_NSUB = 16
_CHUNK = 128
_EPAD = 163840
_CPT = _EPAD // (_NSC * _NSUB * _CHUNK)
_NPAD = 10240
_RPT = _NPAD // _NSUB

_mesh = plsc.VectorSubcoreMesh(core_axis_name="c", subcore_axis_name="s")


@functools.partial(
    pl.kernel,
    out_type=jax.ShapeDtypeStruct((_NSC, _NPAD, _D), jnp.float32),
    mesh=_mesh,
    scratch_types=[
        pltpu.VMEM((_CPT, _CHUNK), jnp.int32),
        pltpu.VMEM((_CHUNK, _D), jnp.float32),
        pltpu.SemaphoreType.DMA,
        pltpu.SemaphoreType.DMA,
        pltpu.VMEM_SHARED((_NPAD, _D), jnp.float32),
    ],
)
def _sc_degree(dst_hbm, out_hbm, dst_v, buf, sem, ssem, acc_sh):
    cid = lax.axis_index("c")
    sid = lax.axis_index("s")
    tile = cid * _NSUB + sid
    pltpu.async_copy(dst_hbm.at[pl.ds(tile * _CPT, _CPT)], dst_v, sem)

    @pl.loop(0, _CHUNK, unroll=4)
    def _(r):
        for k in range(_D // 16):
            buf[r, pl.ds(k * 16, 16)] = jnp.zeros((16,), jnp.float32)

    @pl.loop(0, _RPT, step=_CHUNK)
    def _(r):
        pltpu.sync_copy(buf, acc_sh.at[pl.ds(sid * _RPT + r, _CHUNK)])

    @pl.loop(0, _CHUNK, unroll=8)
    def _(r):
        buf[r, pl.ds(0, 16)] = jnp.full((16,), 1.0, jnp.float32)

    pltpu.make_async_copy(dst_hbm.at[pl.ds(tile * _CPT, _CPT)], dst_v, sem).wait()
    plsc.subcore_barrier()

    pltpu.async_copy(buf, acc_sh.at[dst_v.at[0]], ssem, add=True)

    @pl.loop(1, _CPT)
    def _(j):
        pltpu.async_copy(buf, acc_sh.at[dst_v.at[j]], ssem, add=True)
        pltpu.make_async_copy(buf, acc_sh.at[dst_v.at[j - 1]], ssem).wait()

    pltpu.make_async_copy(buf, acc_sh.at[dst_v.at[_CPT - 1]], ssem).wait()
    plsc.subcore_barrier()
    pltpu.sync_copy(
        acc_sh.at[pl.ds(sid * _RPT, _RPT)],
        out_hbm.at[cid, pl.ds(sid * _RPT, _RPT)],
    )


@functools.partial(
    pl.kernel,
    out_type=jax.ShapeDtypeStruct((_NSC, _NPAD, _D), jnp.float32),
    mesh=_mesh,
    scratch_types=[
        pltpu.VMEM((_CPT, _CHUNK), jnp.int32),
        pltpu.VMEM((_CPT, _CHUNK), jnp.int32),
        [pltpu.VMEM((_CHUNK, _D), jnp.float32)] * 2,
        [pltpu.SemaphoreType.DMA] * 2,
        [pltpu.SemaphoreType.DMA] * 2,
        pltpu.SemaphoreType.DMA,
        pltpu.VMEM_SHARED((_NPAD, _D), jnp.float32),
    ],
)
def _sc_segment_sum(u_hbm, src_hbm, dst_hbm, out_hbm,
                    src_v, dst_v, bufs, gsems, ssems, sem, acc_sh):
    cid = lax.axis_index("c")
    sid = lax.axis_index("s")
    tile = cid * _NSUB + sid
    pltpu.async_copy(src_hbm.at[pl.ds(tile * _CPT, _CPT)], src_v, sem).wait()
    pltpu.async_copy(u_hbm.at[src_v.at[0]], bufs[0], gsems[0])
    pltpu.async_copy(dst_hbm.at[pl.ds(tile * _CPT, _CPT)], dst_v, sem)

    @pl.loop(0, _CHUNK, unroll=4)
    def _(r):
        for k in range(_D // 16):
            bufs[1][r, pl.ds(k * 16, 16)] = jnp.zeros((16,), jnp.float32)

    @pl.loop(0, _RPT, step=_CHUNK)
    def _(r):
        pltpu.sync_copy(bufs[1], acc_sh.at[pl.ds(sid * _RPT + r, _CHUNK)])

    pltpu.make_async_copy(dst_hbm.at[pl.ds(tile * _CPT, _CPT)], dst_v, sem).wait()
    pltpu.async_copy(u_hbm.at[src_v.at[1]], bufs[1], gsems[1])
    plsc.subcore_barrier()

    @pl.loop(0, _CPT, step=2)
    def _(j):
        for t in range(2):
            pltpu.make_async_copy(
                u_hbm.at[src_v.at[j + t]], bufs[t], gsems[t]).wait()
            pltpu.async_copy(bufs[t], acc_sh.at[dst_v.at[j + t]], ssems[t],
                             add=True)
        for t in range(2):
            @pl.when(j + t + 2 < _CPT)
            def _(t=t, j=j):
                pltpu.make_async_copy(
                    bufs[t], acc_sh.at[dst_v.at[j + t]], ssems[t]).wait()
                pltpu.async_copy(
                    u_hbm.at[src_v.at[j + t + 2]], bufs[t], gsems[t])

    for t in range(2):
        pltpu.make_async_copy(bufs[t], acc_sh.at[dst_v.at[0]], ssems[t]).wait()
    plsc.subcore_barrier()
    pltpu.sync_copy(
        acc_sh.at[pl.ds(sid * _RPT, _RPT)],
        out_hbm.at[cid, pl.ds(sid * _RPT, _RPT)],
    )


def _tc_first(hist, x, W1):
    def body(hist_r, x_r, w_r, dis_o, u_o):
        deg = hist_r[0, :_N, 0:1] + hist_r[1, :_N, 0:1] + 1.0
        dis = lax.rsqrt(deg)
        dis_o[...] = dis
        res = jnp.dot(x_r[:, :64], w_r[...],
                      preferred_element_type=jnp.float32) * dis
        u_o[...] = jnp.concatenate(
            [res, jnp.zeros((_N, _D - res.shape[1]), jnp.float32)], axis=1)

    return pl.pallas_call(
        body,
        out_shape=(
            jax.ShapeDtypeStruct((_N, 1), jnp.float32),
            jax.ShapeDtypeStruct((_N, _D), jnp.float32),
        ),
    )(hist, x, W1)


def _tc_mid(y, u, dis, b, W):
    din, dout = W.shape

    def body(y_r, u_r, dis_r, b_r, w_r, u_o):
        s = y_r[0, :_N, :din] + y_r[1, :_N, :din] + u_r[:, :din]
        h = jax.nn.relu(dis_r[...] * s + b_r[...])
        res = jnp.dot(h, w_r[...], preferred_element_type=jnp.float32) * dis_r[...]
        if dout < _D:
            res = jnp.concatenate(
                [res, jnp.zeros((_N, _D - dout), jnp.float32)], axis=1)
        u_o[...] = res

    return pl.pallas_call(
        body,
        out_shape=jax.ShapeDtypeStruct((_N, _D), jnp.float32),
    )(y, u, dis, b, W)


def _tc_epilogue(y, u, dis, b, batch2d, Wl1, bl1, Wl2, bl2):
    def body(y_r, u_r, dis_r, b_r, bat_r, wl1_r, bl1_r, wl2_r, bl2_r, o):
        s = y_r[0, :_N, :] + y_r[1, :_N, :] + u_r[...]
        h = jax.nn.relu(dis_r[...] * s + b_r[...])
        gid = lax.broadcasted_iota(jnp.int32, (1, _G), 1)
        onehot = (bat_r[...] == gid).astype(jnp.float32)
        sums = lax.dot_general(onehot, h, (((0,), (0,)), ((), ())),
                               preferred_element_type=jnp.float32)
        cnt = jnp.sum(onehot, axis=0)[:, None]
        g = sums / jnp.clip(cnt, 1.0, None)
        g = jax.nn.relu(jnp.dot(g, wl1_r[...],
                                preferred_element_type=jnp.float32) + bl1_r[...])
        o[...] = jnp.dot(g, wl2_r[...],
                         preferred_element_type=jnp.float32) + bl2_r[...]

    return pl.pallas_call(
        body,
        out_shape=jax.ShapeDtypeStruct((_G, Wl2.shape[1]), jnp.float32),
    )(y, u, dis, b, batch2d, Wl1, bl1, Wl2, bl2)


def kernel(x, edge_index, batch, W1, b1, W2, b2, W3, b3, W4, b4, W5, b5,
           W6, b6, Wl1, bl1, Wl2, bl2):
    pad = _EPAD - _E
    pad_iota = jnp.arange(pad, dtype=jnp.int32)
    src = jnp.concatenate([edge_index[0], pad_iota % _N])
    dst = jnp.concatenate([edge_index[1], _N + pad_iota % (_NPAD - _N)])
    src = src.reshape(_EPAD // _CHUNK, _CHUNK)
    dst = dst.reshape(_EPAD // _CHUNK, _CHUNK)

    hist = _sc_degree(dst)
    dis, u = _tc_first(hist, x, W1)

    layers = [(b1, W2), (b2, W3), (b3, W4), (b4, W5), (b5, W6)]
    for b, Wn in layers:
        y = _sc_segment_sum(u, src, dst)
        u = _tc_mid(y, u, dis, b, Wn)

    y = _sc_segment_sum(u, src, dst)
    return _tc_epilogue(y, u, dis, b6, batch.reshape(_N, 1),
                        Wl1, bl1, Wl2, bl2)


# speedup vs baseline: 18.8051x; 1.1943x over previous
import functools

import jax
import jax.numpy as jnp
from jax import lax
from jax.experimental import pallas as pl
from jax.experimental.pallas import tpu as pltpu
from jax.experimental.pallas import tpu_sc as plsc

_N = 10000
_E = 160000
_G = 32
_D = 128

_NSC = 2
_NSUB = 16
_CHUNK = 128
_EPAD = 163840
_CPT = _EPAD // (_NSC * _NSUB * _CHUNK)
_NPAD = 10240
_RPT = _NPAD // _NSUB

_mesh = plsc.VectorSubcoreMesh(core_axis_name="c", subcore_axis_name="s")
_cp = pltpu.CompilerParams(use_tc_tiling_on_sc=False)


@functools.partial(
    pl.kernel,
    out_type=jax.ShapeDtypeStruct((_NSC, _NPAD, 16), jnp.float32),
    mesh=_mesh,
    compiler_params=_cp,
    scratch_types=[
        pltpu.VMEM((_CPT, _CHUNK), jnp.int32),
        pltpu.VMEM((_CHUNK, 16), jnp.float32),
        pltpu.SemaphoreType.DMA,
        pltpu.SemaphoreType.DMA,
        pltpu.VMEM_SHARED((_NPAD, 16), jnp.float32),
    ],
)
def _sc_degree(dst_hbm, ones_hbm, zeros_hbm, out_hbm, dst_v, ones_v, sem, ssem,
               acc_sh):
    cid = lax.axis_index("c")
    sid = lax.axis_index("s")
    tile = cid * _NSUB + sid
    pltpu.async_copy(dst_hbm.at[pl.ds(tile * _CPT, _CPT)], dst_v, sem)
    pltpu.sync_copy(ones_hbm, ones_v)
    pltpu.sync_copy(zeros_hbm, acc_sh.at[pl.ds(sid * _RPT, _RPT)])
    pltpu.make_async_copy(dst_hbm.at[pl.ds(tile * _CPT, _CPT)], dst_v, sem).wait()
    plsc.subcore_barrier()

    pltpu.async_copy(ones_v, acc_sh.at[dst_v.at[0]], ssem, add=True)

    @pl.loop(1, _CPT)
    def _(j):
        pltpu.async_copy(ones_v, acc_sh.at[dst_v.at[j]], ssem, add=True)
        pltpu.make_async_copy(ones_v, acc_sh.at[dst_v.at[j - 1]], ssem).wait()

    pltpu.make_async_copy(ones_v, acc_sh.at[dst_v.at[_CPT - 1]], ssem).wait()
    plsc.subcore_barrier()
    pltpu.sync_copy(
        acc_sh.at[pl.ds(sid * _RPT, _RPT)],
        out_hbm.at[cid, pl.ds(sid * _RPT, _RPT)],
    )


@functools.cache
def _make_seg(dout):
    @functools.partial(
        pl.kernel,
        out_type=jax.ShapeDtypeStruct((_NSC, _NPAD, dout), jnp.float32),
        mesh=_mesh,
        compiler_params=_cp,
        scratch_types=[
            pltpu.VMEM((_CPT, _CHUNK), jnp.int32),
            pltpu.VMEM((_CPT, _CHUNK), jnp.int32),
            [pltpu.VMEM((_CHUNK, dout), jnp.float32)] * 2,
            [pltpu.SemaphoreType.DMA] * 2,
            [pltpu.SemaphoreType.DMA] * 2,
            pltpu.SemaphoreType.DMA,
            pltpu.VMEM_SHARED((_NPAD, dout), jnp.float32),
        ],
    )
    def seg(u_hbm, src_hbm, dst_hbm, zeros_hbm, out_hbm,
            src_v, dst_v, bufs, gsems, ssems, sem, acc_sh):
        cid = lax.axis_index("c")
        sid = lax.axis_index("s")
        tile = cid * _NSUB + sid
        pltpu.async_copy(src_hbm.at[pl.ds(tile * _CPT, _CPT)], src_v, sem).wait()
        pltpu.async_copy(u_hbm.at[src_v.at[0]], bufs[0], gsems[0])
        pltpu.async_copy(dst_hbm.at[pl.ds(tile * _CPT, _CPT)], dst_v, sem)
        pltpu.sync_copy(zeros_hbm, acc_sh.at[pl.ds(sid * _RPT, _RPT)])
        pltpu.make_async_copy(dst_hbm.at[pl.ds(tile * _CPT, _CPT)], dst_v,
                              sem).wait()
        pltpu.async_copy(u_hbm.at[src_v.at[1]], bufs[1], gsems[1])
        plsc.subcore_barrier()

        @pl.loop(0, _CPT, step=2)
        def _(j):
            for t in range(2):
                pltpu.make_async_copy(
                    u_hbm.at[src_v.at[j + t]], bufs[t], gsems[t]).wait()
                pltpu.async_copy(bufs[t], acc_sh.at[dst_v.at[j + t]], ssems[t],
                                 add=True)
            for t in range(2):
                @pl.when(j + t + 2 < _CPT)
                def _(t=t, j=j):
                    pltpu.make_async_copy(
                        bufs[t], acc_sh.at[dst_v.at[j + t]], ssems[t]).wait()
                    pltpu.async_copy(
                        u_hbm.at[src_v.at[j + t + 2]], bufs[t], gsems[t])

        for t in range(2):
            pltpu.make_async_copy(bufs[t], acc_sh.at[dst_v.at[0]],
                                  ssems[t]).wait()
        plsc.subcore_barrier()
        pltpu.sync_copy(
            acc_sh.at[pl.ds(sid * _RPT, _RPT)],
            out_hbm.at[cid, pl.ds(sid * _RPT, _RPT)],
        )

    return seg


def _tc_first(hist, x, W1):
    def body(hist_r, x_r, w_r, dis_o, u_o):
        deg = hist_r[0, :_N, 0:1] + hist_r[1, :_N, 0:1] + 1.0
        dis = lax.rsqrt(deg)
        dis_o[...] = dis
        u_o[...] = jnp.dot(x_r[:, :64], w_r[...],
                           preferred_element_type=jnp.float32) * dis

    return pl.pallas_call(
        body,
        out_shape=(
            jax.ShapeDtypeStruct((_N, 1), jnp.float32),
            jax.ShapeDtypeStruct((_N, W1.shape[1]), jnp.float32),
        ),
    )(hist, x, W1)


def _tc_mid(y, u, dis, b, W):
    def body(y_r, u_r, dis_r, b_r, w_r, u_o):
        s = y_r[0, :_N, :] + y_r[1, :_N, :] + u_r[...]
        h = jax.nn.relu(dis_r[...] * s + b_r[...])
        u_o[...] = jnp.dot(h, w_r[...],
                           preferred_element_type=jnp.float32) * dis_r[...]

    return pl.pallas_call(
        body,
        out_shape=jax.ShapeDtypeStruct((_N, W.shape[1]), jnp.float32),
    )(y, u, dis, b, W)


def _tc_epilogue(y, u, dis, b, batch2d, Wl1, bl1, Wl2, bl2):
    def body(y_r, u_r, dis_r, b_r, bat_r, wl1_r, bl1_r, wl2_r, bl2_r, o):
        s = y_r[0, :_N, :] + y_r[1, :_N, :] + u_r[...]
        h = jax.nn.relu(dis_r[...] * s + b_r[...])
        gid = lax.broadcasted_iota(jnp.int32, (1, _G), 1)
        onehot = (bat_r[...] == gid).astype(jnp.float32)
        sums = lax.dot_general(onehot, h, (((0,), (0,)), ((), ())),
                               preferred_element_type=jnp.float32)
        cnt = jnp.sum(onehot, axis=0)[:, None]
        g = sums / jnp.clip(cnt, 1.0, None)
        g = jax.nn.relu(jnp.dot(g, wl1_r[...],
                                preferred_element_type=jnp.float32) + bl1_r[...])
        o[...] = jnp.dot(g, wl2_r[...],
                         preferred_element_type=jnp.float32) + bl2_r[...]

    return pl.pallas_call(
        body,
        out_shape=jax.ShapeDtypeStruct((_G, Wl2.shape[1]), jnp.float32),
    )(y, u, dis, b, batch2d, Wl1, bl1, Wl2, bl2)


def kernel(x, edge_index, batch, W1, b1, W2, b2, W3, b3, W4, b4, W5, b5,
           W6, b6, Wl1, bl1, Wl2, bl2):
    pad = _EPAD - _E
    pad_iota = jnp.arange(pad, dtype=jnp.int32)
    src = jnp.concatenate([edge_index[0], pad_iota % _N])
    dst = jnp.concatenate([edge_index[1], _N + pad_iota % (_NPAD - _N)])
    src = src.reshape(_EPAD // _CHUNK, _CHUNK)
    dst = dst.reshape(_EPAD // _CHUNK, _CHUNK)

    hist = _sc_degree(dst, jnp.ones((_CHUNK, 16), jnp.float32),
                      jnp.zeros((_RPT, 16), jnp.float32))
    dis, u = _tc_first(hist, x, W1)

    layers = [(b1, W2), (b2, W3), (b3, W4), (b4, W5), (b5, W6), (b6, None)]
    for b, Wn in layers:
        dout = u.shape[1]
        y = _make_seg(dout)(u, src, dst, jnp.zeros((_RPT, dout), jnp.float32))
        if Wn is None:
            return _tc_epilogue(y, u, dis, b, batch.reshape(_N, 1),
                                Wl1, bl1, Wl2, bl2)
        u = _tc_mid(y, u, dis, b, Wn)


# speedup vs baseline: 20.8944x; 1.1111x over previous
import functools

import jax
import jax.numpy as jnp
from jax import lax
from jax.experimental import pallas as pl
from jax.experimental.pallas import tpu as pltpu
from jax.experimental.pallas import tpu_sc as plsc

_N = 10000
_E = 160000
_G = 32
_D = 128

_NSC = 2
_NSUB = 16
_CHUNK = 128
_EPAD = 163840
_CPT = _EPAD // (_NSC * _NSUB * _CHUNK)
_NPAD = 10240
_RPT = _NPAD // _NSUB

_mesh = plsc.VectorSubcoreMesh(core_axis_name="c", subcore_axis_name="s")
_cp = pltpu.CompilerParams(use_tc_tiling_on_sc=False)


@functools.partial(
    pl.kernel,
    out_type=jax.ShapeDtypeStruct((_NSC, _NPAD, 16), jnp.float32),
    mesh=_mesh,
    compiler_params=_cp,
    scratch_types=[
        pltpu.VMEM((_CPT, _CHUNK), jnp.int32),
        pltpu.VMEM((_CHUNK, 16), jnp.float32),
        pltpu.SemaphoreType.DMA,
        pltpu.SemaphoreType.DMA,
        pltpu.VMEM_SHARED((_NPAD, 16), jnp.float32),
    ],
)
def _sc_degree(dst_hbm, ones_hbm, zeros_hbm, out_hbm, dst_v, ones_v, sem, ssem,
               acc_sh):
    cid = lax.axis_index("c")
    sid = lax.axis_index("s")
    tile = cid * _NSUB + sid
    pltpu.async_copy(dst_hbm.at[pl.ds(tile * _CPT, _CPT)], dst_v, sem)
    pltpu.sync_copy(ones_hbm, ones_v)
    pltpu.sync_copy(zeros_hbm, acc_sh.at[pl.ds(sid * _RPT, _RPT)])
    pltpu.make_async_copy(dst_hbm.at[pl.ds(tile * _CPT, _CPT)], dst_v, sem).wait()
    plsc.subcore_barrier()

    pltpu.async_copy(ones_v, acc_sh.at[dst_v.at[0]], ssem, add=True)

    @pl.loop(1, _CPT)
    def _(j):
        pltpu.async_copy(ones_v, acc_sh.at[dst_v.at[j]], ssem, add=True)
        pltpu.make_async_copy(ones_v, acc_sh.at[dst_v.at[j - 1]], ssem).wait()

    pltpu.make_async_copy(ones_v, acc_sh.at[dst_v.at[_CPT - 1]], ssem).wait()
    plsc.subcore_barrier()
    pltpu.sync_copy(
        acc_sh.at[pl.ds(sid * _RPT, _RPT)],
        out_hbm.at[cid, pl.ds(sid * _RPT, _RPT)],
    )


@functools.cache
def _make_seg(dout):
    nbuf = 2 if dout > 96 else 4

    @functools.partial(
        pl.kernel,
        out_type=jax.ShapeDtypeStruct((_NSC, _NPAD, dout), jnp.float32),
        mesh=_mesh,
        compiler_params=_cp,
        scratch_types=[
            pltpu.VMEM((_CPT, _CHUNK), jnp.int32),
            pltpu.VMEM((_CPT, _CHUNK), jnp.int32),
            [pltpu.VMEM((_CHUNK, dout), jnp.float32)] * nbuf,
            [pltpu.SemaphoreType.DMA] * nbuf,
            [pltpu.SemaphoreType.DMA] * nbuf,
            pltpu.SemaphoreType.DMA,
            pltpu.VMEM_SHARED((_NPAD, dout), jnp.float32),
        ],
    )
    def seg(u_hbm, src_hbm, dst_hbm, zeros_hbm, out_hbm,
            src_v, dst_v, bufs, gsems, ssems, sem, acc_sh):
        cid = lax.axis_index("c")
        sid = lax.axis_index("s")
        tile = cid * _NSUB + sid
        pltpu.async_copy(src_hbm.at[pl.ds(tile * _CPT, _CPT)], src_v, sem).wait()
        pltpu.async_copy(u_hbm.at[src_v.at[0]], bufs[0], gsems[0])
        pltpu.async_copy(dst_hbm.at[pl.ds(tile * _CPT, _CPT)], dst_v, sem)
        pltpu.sync_copy(zeros_hbm.at[pl.ds(sid * _RPT, _RPT)],
                        acc_sh.at[pl.ds(sid * _RPT, _RPT)])
        pltpu.make_async_copy(dst_hbm.at[pl.ds(tile * _CPT, _CPT)], dst_v,
                              sem).wait()
        pltpu.async_copy(u_hbm.at[src_v.at[1]], bufs[1], gsems[1])
        plsc.subcore_barrier()

        if nbuf == 2:
            @pl.loop(0, _CPT, step=2)
            def _(j):
                for t in range(2):
                    pltpu.make_async_copy(
                        u_hbm.at[src_v.at[j + t]], bufs[t], gsems[t]).wait()
                    pltpu.async_copy(bufs[t], acc_sh.at[dst_v.at[j + t]],
                                     ssems[t], add=True)
                for t in range(2):
                    @pl.when(j + t + 2 < _CPT)
                    def _(t=t, j=j):
                        pltpu.make_async_copy(
                            bufs[t], acc_sh.at[dst_v.at[j + t]],
                            ssems[t]).wait()
                        pltpu.async_copy(
                            u_hbm.at[src_v.at[j + t + 2]], bufs[t], gsems[t])
        else:
            @pl.loop(0, _CPT, step=4)
            def _(j):
                for t in range(4):
                    t2 = (t + 2) % 4
                    pltpu.make_async_copy(
                        u_hbm.at[src_v.at[j + t]], bufs[t], gsems[t]).wait()
                    pltpu.async_copy(bufs[t], acc_sh.at[dst_v.at[j + t]],
                                     ssems[t], add=True)

                    @pl.when(j + t + 2 < _CPT)
                    def _(t=t, t2=t2, j=j):
                        @pl.when(j + t >= 2)
                        def _(t2=t2, j=j, t=t):
                            pltpu.make_async_copy(
                                bufs[t2], acc_sh.at[dst_v.at[j + t]],
                                ssems[t2]).wait()

                        pltpu.async_copy(
                            u_hbm.at[src_v.at[j + t + 2]], bufs[t2],
                            gsems[t2])

        for t in range(nbuf):
            pltpu.make_async_copy(bufs[t], acc_sh.at[dst_v.at[0]],
                                  ssems[t]).wait()
        plsc.subcore_barrier()
        pltpu.sync_copy(
            acc_sh.at[pl.ds(sid * _RPT, _RPT)],
            out_hbm.at[cid, pl.ds(sid * _RPT, _RPT)],
        )

    return seg


def _tc_first(hist, x, W1):
    def body(hist_r, x_r, w_r, dis_o, u_o):
        deg = hist_r[0, :_N, 0:1] + hist_r[1, :_N, 0:1] + 1.0
        dis = lax.rsqrt(deg)
        dis_o[...] = dis
        u_o[...] = jnp.dot(x_r[:, :64], w_r[...],
                           preferred_element_type=jnp.float32) * dis

    return pl.pallas_call(
        body,
        out_shape=(
            jax.ShapeDtypeStruct((_N, 1), jnp.float32),
            jax.ShapeDtypeStruct((_N, W1.shape[1]), jnp.float32),
        ),
    )(hist, x, W1)


def _tc_mid(y, u, dis, b, W):
    def body(y_r, u_r, dis_r, b_r, w_r, u_o):
        s = y_r[0, :_N, :] + y_r[1, :_N, :] + u_r[...]
        h = jax.nn.relu(dis_r[...] * s + b_r[...])
        u_o[...] = jnp.dot(h, w_r[...],
                           preferred_element_type=jnp.float32) * dis_r[...]

    return pl.pallas_call(
        body,
        out_shape=jax.ShapeDtypeStruct((_N, W.shape[1]), jnp.float32),
    )(y, u, dis, b, W)


def _tc_epilogue(y, u, dis, b, batch2d, Wl1, bl1, Wl2, bl2):
    def body(y_r, u_r, dis_r, b_r, bat_r, wl1_r, bl1_r, wl2_r, bl2_r, o):
        s = y_r[0, :_N, :] + y_r[1, :_N, :] + u_r[...]
        h = jax.nn.relu(dis_r[...] * s + b_r[...])
        gid = lax.broadcasted_iota(jnp.int32, (1, _G), 1)
        onehot = (bat_r[...] == gid).astype(jnp.float32)
        sums = lax.dot_general(onehot, h, (((0,), (0,)), ((), ())),
                               preferred_element_type=jnp.float32)
        cnt = jnp.sum(onehot, axis=0)[:, None]
        g = sums / jnp.clip(cnt, 1.0, None)
        g = jax.nn.relu(jnp.dot(g, wl1_r[...],
                                preferred_element_type=jnp.float32) + bl1_r[...])
        o[...] = jnp.dot(g, wl2_r[...],
                         preferred_element_type=jnp.float32) + bl2_r[...]

    return pl.pallas_call(
        body,
        out_shape=jax.ShapeDtypeStruct((_G, Wl2.shape[1]), jnp.float32),
    )(y, u, dis, b, batch2d, Wl1, bl1, Wl2, bl2)


def kernel(x, edge_index, batch, W1, b1, W2, b2, W3, b3, W4, b4, W5, b5,
           W6, b6, Wl1, bl1, Wl2, bl2):
    pad = _EPAD - _E
    pad_iota = jnp.arange(pad, dtype=jnp.int32)
    src = jnp.concatenate([edge_index[0], pad_iota % _N])
    dst = jnp.concatenate([edge_index[1], _N + pad_iota % (_NPAD - _N)])
    src = src.reshape(_EPAD // _CHUNK, _CHUNK)
    dst = dst.reshape(_EPAD // _CHUNK, _CHUNK)

    hist = _sc_degree(dst, jnp.ones((_CHUNK, 16), jnp.float32),
                      jnp.zeros((_RPT, 16), jnp.float32))
    dis, u = _tc_first(hist, x, W1)

    layers = [(b1, W2), (b2, W3), (b3, W4), (b4, W5), (b5, W6), (b6, None)]
    for b, Wn in layers:
        dout = u.shape[1]
        y = _make_seg(dout)(u, src, dst,
                            jnp.zeros((_NPAD, dout), jnp.float32))
        if Wn is None:
            return _tc_epilogue(y, u, dis, b, batch.reshape(_N, 1),
                                Wl1, bl1, Wl2, bl2)
        u = _tc_mid(y, u, dis, b, Wn)
